# speedup vs baseline: 1.0513x; 1.0513x over previous
import functools

import jax
import jax.numpy as jnp
from jax.experimental import pallas as pl
from jax.experimental.pallas import tpu as pltpu

H = 256


def _ln(x, g, b, eps=1e-5):
    m = jnp.mean(x, axis=-1, keepdims=True)
    v = jnp.mean((x - m) ** 2, axis=-1, keepdims=True)
    return (x - m) * jax.lax.rsqrt(v + eps) * g + b


def _edge_mlp_body(ef_ref, rf_ref, nf_ref, We_ref, be_ref, out_ref):
    acc = jnp.dot(ef_ref[...], We_ref[0:H, :], preferred_element_type=jnp.float32)
    acc += jnp.dot(rf_ref[...], We_ref[H:2 * H, :], preferred_element_type=jnp.float32)
    acc += jnp.dot(nf_ref[...], We_ref[2 * H:3 * H, :], preferred_element_type=jnp.float32)
    out_ref[...] = jnp.maximum(acc + be_ref[...], 0.0)


def _edge_mlp(ef, rf, nf, We, be, block_rows):
    E = ef.shape[0]
    assert E % block_rows == 0
    grid = (E // block_rows,)
    row_spec = pl.BlockSpec((block_rows, H), lambda i: (i, 0))
    full_w = pl.BlockSpec((3 * H, H), lambda i: (0, 0))
    vec = pl.BlockSpec((H,), lambda i: (0,))
    return pl.pallas_call(
        _edge_mlp_body,
        grid=grid,
        in_specs=[row_spec, row_spec, row_spec, full_w, vec],
        out_specs=row_spec,
        out_shape=jax.ShapeDtypeStruct((E, H), jnp.float32),
    )(ef, rf, nf, We, be)


def _node_mlp_block(nf, agg, W1_ref, b1_ref, g1_ref, bt1_ref, W2_ref, b2_ref,
                    g2_ref, bt2_ref):
    m = jnp.dot(nf, W1_ref[0:H, :], preferred_element_type=jnp.float32)
    m += jnp.dot(agg, W1_ref[H:2 * H, :], preferred_element_type=jnp.float32)
    h = jnp.maximum(_ln(m + b1_ref[...], g1_ref[...], bt1_ref[...]), 0.0)
    h2 = jnp.dot(h, W2_ref[...], preferred_element_type=jnp.float32) + b2_ref[...]
    return jnp.maximum(_ln(h2, g2_ref[...], bt2_ref[...]), 0.0)


def _node_edge_body(nf1_ref, agg_ref, ef_ref, rf_ref,
                    W1_ref, b1_ref, g1_ref, bt1_ref, W2_ref, b2_ref, g2_ref,
                    bt2_ref, We_ref, be_ref, out_ref):
    n1 = _node_mlp_block(nf1_ref[...], agg_ref[...], W1_ref, b1_ref, g1_ref,
                         bt1_ref, W2_ref, b2_ref, g2_ref, bt2_ref)
    acc = jnp.dot(ef_ref[...], We_ref[0:H, :], preferred_element_type=jnp.float32)
    acc += jnp.dot(rf_ref[...], We_ref[H:2 * H, :], preferred_element_type=jnp.float32)
    acc += jnp.dot(n1, We_ref[2 * H:3 * H, :], preferred_element_type=jnp.float32)
    out_ref[...] = jnp.maximum(acc + be_ref[...], 0.0)


def _node_edge_mlp(nf1, agg, ef, rf, W1, b1, g1, bt1, W2, b2, g2, bt2, We, be,
                   block_rows):
    S = nf1.shape[0]
    assert S % block_rows == 0
    grid = (S // block_rows,)
    row_spec = pl.BlockSpec((block_rows, H), lambda i: (i, 0))
    w2h = pl.BlockSpec((2 * H, H), lambda i: (0, 0))
    w1h = pl.BlockSpec((H, H), lambda i: (0, 0))
    w3h = pl.BlockSpec((3 * H, H), lambda i: (0, 0))
    vec = pl.BlockSpec((H,), lambda i: (0,))
    return pl.pallas_call(
        _node_edge_body,
        grid=grid,
        in_specs=[row_spec, row_spec, row_spec, row_spec,
                  w2h, vec, vec, vec, w1h, vec, vec, vec, w3h, vec],
        out_specs=row_spec,
        out_shape=jax.ShapeDtypeStruct((S, H), jnp.float32),
    )(nf1, agg, ef, rf, W1, b1, g1, bt1, W2, b2, g2, bt2, We, be)


def _node_body(nf_ref, agg_ref, W1_ref, b1_ref, g1_ref, bt1_ref, W2_ref,
               b2_ref, g2_ref, bt2_ref, out_ref):
    out_ref[...] = _node_mlp_block(nf_ref[...], agg_ref[...], W1_ref, b1_ref,
                                   g1_ref, bt1_ref, W2_ref, b2_ref, g2_ref,
                                   bt2_ref)


def _node_mlp(nf, agg, W1, b1, g1, bt1, W2, b2, g2, bt2, block_rows):
    S = nf.shape[0]
    assert S % block_rows == 0
    grid = (S // block_rows,)
    row_spec = pl.BlockSpec((block_rows, H), lambda i: (i, 0))
    w2h = pl.BlockSpec((2 * H, H), lambda i: (0, 0))
    w1h = pl.BlockSpec((H, H), lambda i: (0, 0))
    vec = pl.BlockSpec((H,), lambda i: (0,))
    return pl.pallas_call(
        _node_body,
        grid=grid,
        in_specs=[row_spec, row_spec, w2h, vec, vec, vec, w1h, vec, vec, vec],
        out_specs=row_spec,
        out_shape=jax.ShapeDtypeStruct((S, H), jnp.float32),
    )(nf, agg, W1, b1, g1, bt1, W2, b2, g2, bt2)


def _segment_sum(x, dst, num_segments):
    return jax.ops.segment_sum(x, dst, num_segments=num_segments)


def kernel(n_feat_0, n_feat_1, n_feat_2, e_feat_1, e_feat_2, r_feat_1,
           r_feat_2, dst_1, dst_2, We_1, be_1, W1_1, b1_1, g1_1, bt1_1, W2_1,
           b2_1, g2_1, bt2_1, We_2, be_2, W1_2, b1_2, g1_2, bt1_2, W2_2, b2_2,
           g2_2, bt2_2):
    N0, N1, N2 = n_feat_0.shape[0], n_feat_1.shape[0], n_feat_2.shape[0]

    e_repr_2 = _edge_mlp(e_feat_2, r_feat_2, n_feat_2, We_2, be_2,
                         block_rows=1000)
    agg_2 = _segment_sum(e_repr_2, dst_2, N1)
    e_repr_1 = _node_edge_mlp(n_feat_1, agg_2, e_feat_1, r_feat_1,
                              W1_2, b1_2, g1_2, bt1_2, W2_2, b2_2, g2_2, bt2_2,
                              We_1, be_1, block_rows=1000)
    agg_1 = _segment_sum(e_repr_1, dst_1, N0)
    n0 = _node_mlp(n_feat_0, agg_1, W1_1, b1_1, g1_1, bt1_1, W2_1, b2_1,
                   g2_1, bt2_1, block_rows=1000)
    return n0
